# baseline (speedup 1.0000x reference)
import functools

import jax
import jax.numpy as jnp
from jax import lax
from jax.experimental import pallas as pl
from jax.experimental.pallas import tpu as pltpu
from jax.experimental.pallas import tpu_sc as plsc

_N = 10000
_E = 320000
_NPAD = 10112
_NW = 32
_CH = 128
_KCH = 80
_EPAD = _NW * _KCH * _CH
_EPW = _KCH * _CH
_RPT = _NPAD // 16
_TCH = _EPAD // _CH
_SPS = _TCH // 16
_K0 = 112


def _sc_mesh():
    return plsc.VectorSubcoreMesh(core_axis_name="c", subcore_axis_name="s")


@functools.partial(
    pl.kernel,
    out_type=jax.ShapeDtypeStruct((2, _NPAD, 16), jnp.float32),
    mesh=_sc_mesh(),
    scratch_types=[
        pltpu.VMEM((_KCH, _CH), jnp.int32),
        pltpu.VMEM((_CH, 16), jnp.float32),
        pltpu.VMEM_SHARED((_NPAD, 16), jnp.float32),
        pltpu.SemaphoreType.DMA,
    ],
    compiler_params=pltpu.CompilerParams(use_tc_tiling_on_sc=False),
)
def _deg_sc(dst_hbm, ones_hbm, zeros_hbm, out_hbm, idx_v, ones_v, acc_sh,
            ssem):
    c = lax.axis_index("c")
    s = lax.axis_index("s")
    w = s * 2 + c
    pltpu.sync_copy(zeros_hbm, acc_sh.at[pl.ds(s * _RPT, _RPT)])
    pltpu.sync_copy(dst_hbm.at[w], idx_v)
    pltpu.sync_copy(ones_hbm, ones_v)
    plsc.subcore_barrier()

    def body(j, carry):
        pltpu.async_copy(ones_v, acc_sh.at[idx_v.at[j]], ssem, add=True)

        @pl.when(j >= 4)
        def _():
            pltpu.make_async_copy(ones_v, acc_sh.at[idx_v.at[0]],
                                  ssem).wait()

        return carry

    lax.fori_loop(0, _KCH, body, 0)

    def drain(j, carry):
        pltpu.make_async_copy(ones_v, acc_sh.at[idx_v.at[0]], ssem).wait()
        return carry

    lax.fori_loop(0, 4, drain, 0)
    plsc.subcore_barrier()
    pltpu.sync_copy(acc_sh.at[pl.ds(s * _RPT, _RPT)],
                    out_hbm.at[c, pl.ds(s * _RPT, _RPT)])


_IB = 16


@functools.partial(
    pl.kernel,
    out_type=jax.ShapeDtypeStruct((2, _NPAD, 128), jnp.float32),
    mesh=_sc_mesh(),
    scratch_types=[
        pltpu.VMEM((2, _IB, 2, _CH), jnp.int32),
        pltpu.VMEM((2, _CH, 128), jnp.float32),
        pltpu.VMEM_SHARED((_NPAD, 128), jnp.float32),
        pltpu.SemaphoreType.DMA,
        pltpu.SemaphoreType.DMA,
    ],
    compiler_params=pltpu.CompilerParams(use_tc_tiling_on_sc=False),
)
def _scatter128(g_hbm, ei_hbm, zeros_hbm, out_hbm, idx_v, rows_v, acc_sh,
                gs0, gs1):
    c = lax.axis_index("c")
    s = lax.axis_index("s")
    pltpu.sync_copy(zeros_hbm, acc_sh.at[pl.ds(s * _RPT, _RPT)])
    plsc.subcore_barrier()

    base = s * _SPS + c * _K0
    myk = jnp.where(c == 0, _K0, _SPS - _K0)

    def body(j, carry):
        @pl.when(jnp.logical_and(j < myk, j % _IB == 0))
        def _():
            pltpu.sync_copy(ei_hbm.at[pl.ds(base + j, _IB)],
                            idx_v.at[(j // _IB) % 2])

        @pl.when(j < myk)
        def _():
            wj = (j // _IB) % 2
            sj = j % _IB

            @pl.when(j % 2 == 0)
            def _():
                pltpu.async_copy(g_hbm.at[idx_v.at[wj, sj, 0]],
                                 rows_v.at[0], gs0)

            @pl.when(j % 2 == 1)
            def _():
                pltpu.async_copy(g_hbm.at[idx_v.at[wj, sj, 0]],
                                 rows_v.at[1], gs1)

        @pl.when(j >= 1)
        def _():
            jm = j - 1
            wm = (jm // _IB) % 2
            sm = jm % _IB

            @pl.when(jm % 2 == 0)
            def _():
                pltpu.make_async_copy(g_hbm.at[idx_v.at[0, 0, 0]],
                                      rows_v.at[0], gs0).wait()

            @pl.when(jm % 2 == 1)
            def _():
                pltpu.make_async_copy(g_hbm.at[idx_v.at[0, 0, 0]],
                                      rows_v.at[1], gs1).wait()

            pltpu.sync_copy(rows_v.at[jm % 2], acc_sh.at[idx_v.at[wm, sm, 1]],
                            add=True)

        return carry

    lax.fori_loop(0, myk + 1, body, 0)
    plsc.subcore_barrier()
    pltpu.sync_copy(acc_sh.at[pl.ds(s * _RPT, _RPT)],
                    out_hbm.at[c, pl.ds(s * _RPT, _RPT)])


@functools.partial(
    pl.kernel,
    out_type=jax.ShapeDtypeStruct((2, _NPAD, 64), jnp.float32),
    mesh=_sc_mesh(),
    scratch_types=[
        pltpu.VMEM((2, _IB, 2, _CH), jnp.int32),
        pltpu.VMEM((2, _CH, 64), jnp.float32),
        pltpu.VMEM_SHARED((_NPAD, 64), jnp.float32),
        pltpu.SemaphoreType.DMA,
        pltpu.SemaphoreType.DMA,
    ],
    compiler_params=pltpu.CompilerParams(use_tc_tiling_on_sc=False),
)
def _scatter64(g_hbm, ei_hbm, zeros_hbm, out_hbm, idx_v, rows_v, acc_sh,
               gs0, gs1):
    c = lax.axis_index("c")
    s = lax.axis_index("s")
    pltpu.sync_copy(zeros_hbm, acc_sh.at[pl.ds(s * _RPT, _RPT)])
    plsc.subcore_barrier()

    base = s * _SPS + c * _K0
    myk = jnp.where(c == 0, _K0, _SPS - _K0)

    def body(j, carry):
        @pl.when(jnp.logical_and(j < myk, j % _IB == 0))
        def _():
            pltpu.sync_copy(ei_hbm.at[pl.ds(base + j, _IB)],
                            idx_v.at[(j // _IB) % 2])

        @pl.when(j < myk)
        def _():
            wj = (j // _IB) % 2
            sj = j % _IB

            @pl.when(j % 2 == 0)
            def _():
                pltpu.async_copy(g_hbm.at[idx_v.at[wj, sj, 0]],
                                 rows_v.at[0], gs0)

            @pl.when(j % 2 == 1)
            def _():
                pltpu.async_copy(g_hbm.at[idx_v.at[wj, sj, 0]],
                                 rows_v.at[1], gs1)

        @pl.when(j >= 1)
        def _():
            jm = j - 1
            wm = (jm // _IB) % 2
            sm = jm % _IB

            @pl.when(jm % 2 == 0)
            def _():
                pltpu.make_async_copy(g_hbm.at[idx_v.at[0, 0, 0]],
                                      rows_v.at[0], gs0).wait()

            @pl.when(jm % 2 == 1)
            def _():
                pltpu.make_async_copy(g_hbm.at[idx_v.at[0, 0, 0]],
                                      rows_v.at[1], gs1).wait()

            pltpu.sync_copy(rows_v.at[jm % 2], acc_sh.at[idx_v.at[wm, sm, 1]],
                            add=True)

        return carry

    lax.fori_loop(0, myk + 1, body, 0)
    plsc.subcore_barrier()
    pltpu.sync_copy(acc_sh.at[pl.ds(s * _RPT, _RPT)],
                    out_hbm.at[c, pl.ds(s * _RPT, _RPT)])


def _tca_body(x_ref, w1_ref, degp_ref, g1_ref, dinv_ref):
    deg = degp_ref[0, :, 0:1] + degp_ref[1, :, 0:1] + 1.0
    dinv = lax.rsqrt(deg)
    h = jnp.dot(x_ref[...], w1_ref[...], preferred_element_type=jnp.float32)
    g1_ref[...] = dinv * h
    dinv_ref[...] = jnp.broadcast_to(dinv, dinv_ref.shape)


_BLK = 632


def _tc_a(x_pad, W1, degT):
    n_blk = _NPAD // _BLK
    return pl.pallas_call(
        _tca_body,
        grid=(n_blk,),
        in_specs=[
            pl.BlockSpec((_BLK, 128), lambda i: (i, 0)),
            pl.BlockSpec((128, 128), lambda i: (0, 0)),
            pl.BlockSpec((2, _BLK, 16), lambda i: (0, i, 0)),
        ],
        out_specs=[
            pl.BlockSpec((_BLK, 128), lambda i: (i, 0)),
            pl.BlockSpec((_BLK, 8), lambda i: (i, 0)),
        ],
        out_shape=[
            jax.ShapeDtypeStruct((_NPAD, 128), jnp.float32),
            jax.ShapeDtypeStruct((_NPAD, 8), jnp.float32),
        ],
    )(x_pad, W1, degT)


def _tcb_body(acc_ref, g1_ref, dinv_ref, b1_ref, lw1_ref, lb1_ref,
              lw2_ref, lb2_ref, w2_ref, g2_ref):
    dinv = dinv_ref[:, 0:1]
    s = acc_ref[0] + acc_ref[1] + g1_ref[...]
    out1 = jnp.maximum(dinv * s + b1_ref[...], 0.0)
    h2 = jnp.maximum(
        jnp.dot(out1, lw1_ref[...], preferred_element_type=jnp.float32)
        + lb1_ref[...], 0.0)
    h3 = jnp.maximum(
        jnp.dot(h2, lw2_ref[...], preferred_element_type=jnp.float32)
        + lb2_ref[...], 0.0)
    g2_ref[...] = dinv * jnp.dot(h3, w2_ref[...],
                                 preferred_element_type=jnp.float32)


def _tc_b(acc1, g1, dinv8, b1, Lw1, Lb1, Lw2, Lb2, W2):
    n_blk = _NPAD // _BLK
    return pl.pallas_call(
        _tcb_body,
        grid=(n_blk,),
        in_specs=[
            pl.BlockSpec((2, _BLK, 128), lambda i: (0, i, 0)),
            pl.BlockSpec((_BLK, 128), lambda i: (i, 0)),
            pl.BlockSpec((_BLK, 8), lambda i: (i, 0)),
            pl.BlockSpec((1, 128), lambda i: (0, 0)),
            pl.BlockSpec((128, 128), lambda i: (0, 0)),
            pl.BlockSpec((1, 128), lambda i: (0, 0)),
            pl.BlockSpec((128, 128), lambda i: (0, 0)),
            pl.BlockSpec((1, 128), lambda i: (0, 0)),
            pl.BlockSpec((128, 64), lambda i: (0, 0)),
        ],
        out_specs=pl.BlockSpec((_BLK, 64), lambda i: (i, 0)),
        out_shape=jax.ShapeDtypeStruct((_NPAD, 64), jnp.float32),
    )(acc1, g1, dinv8, b1, Lw1, Lb1, Lw2, Lb2, W2)


def _tcc_body(acc_ref, g2_ref, dinv_ref, b2_ref, out_ref):
    dinv = dinv_ref[:, 0:1]
    s = acc_ref[0] + acc_ref[1] + g2_ref[...]
    out_ref[...] = dinv * s + b2_ref[...]


_CBLK = 2000


def _tc_c(acc2, g2, dinv8, b2):
    n_blk = _N // _CBLK
    return pl.pallas_call(
        _tcc_body,
        grid=(n_blk,),
        in_specs=[
            pl.BlockSpec((2, _CBLK, 64), lambda i: (0, i, 0)),
            pl.BlockSpec((_CBLK, 64), lambda i: (i, 0)),
            pl.BlockSpec((_CBLK, 8), lambda i: (i, 0)),
            pl.BlockSpec((1, 64), lambda i: (0, 0)),
        ],
        out_specs=pl.BlockSpec((_CBLK, 64), lambda i: (i, 0)),
        out_shape=jax.ShapeDtypeStruct((_N, 64), jnp.float32),
    )(acc2, g2, dinv8, b2)


def kernel(x, edge_index, W1, b1, Lw1, Lb1, Lw2, Lb2, W2, b2):
    src = edge_index[0]
    dst = edge_index[1]
    pad = jnp.full((_EPAD - _E,), _N, jnp.int32)
    srcf = jnp.concatenate([src, pad])
    dstf = jnp.concatenate([dst, pad])
    dstp = dstf.reshape(_NW, _KCH, _CH)
    ei = jnp.stack([srcf.reshape(_TCH, _CH), dstf.reshape(_TCH, _CH)],
                   axis=1)
    x_pad = jnp.zeros((_NPAD, 128), jnp.float32).at[:_N].set(x)

    ones16 = jnp.ones((_CH, 16), jnp.float32)
    zeros16 = jnp.zeros((_RPT, 16), jnp.float32)
    zeros128 = jnp.zeros((_RPT, 128), jnp.float32)

    degp = _deg_sc(dstp, ones16, zeros16)
    g1, dinv8 = _tc_a(x_pad, W1, degp)
    acc1 = _scatter128(g1, ei, zeros128)
    g2 = _tc_b(acc1, g1, dinv8, b1.reshape(1, 128), Lw1,
               Lb1.reshape(1, 128), Lw2, Lb2.reshape(1, 128), W2)
    zeros64 = jnp.zeros((_RPT, 64), jnp.float32)
    acc2 = _scatter64(g2, ei, zeros64)
    return _tc_c(acc2, g2, dinv8, b2.reshape(1, 64))


# speedup vs baseline: 1.0223x; 1.0223x over previous
import functools

import jax
import jax.numpy as jnp
from jax import lax
from jax.experimental import pallas as pl
from jax.experimental.pallas import tpu as pltpu
from jax.experimental.pallas import tpu_sc as plsc

_N = 10000
_E = 320000
_NPAD = 10112
_NW = 32
_CH = 128
_KCH = 80
_EPAD = _NW * _KCH * _CH
_EPW = _KCH * _CH
_RPT = _NPAD // 16
_TCH = _EPAD // _CH
_SPS = _TCH // 16
_K0 = 112


def _sc_mesh():
    return plsc.VectorSubcoreMesh(core_axis_name="c", subcore_axis_name="s")


@functools.partial(
    pl.kernel,
    out_type=jax.ShapeDtypeStruct((2, _NPAD, 128), jnp.float32),
    mesh=_sc_mesh(),
    scratch_types=[
        pltpu.VMEM((_KCH, _CH), jnp.int32),
        pltpu.VMEM((_CH, 128), jnp.float32),
        pltpu.VMEM_SHARED((_NPAD, 128), jnp.float32),
        pltpu.SemaphoreType.DMA,
    ],
    compiler_params=pltpu.CompilerParams(use_tc_tiling_on_sc=False),
)
def _deg_sc(dst_hbm, ones_hbm, zeros_hbm, out_hbm, idx_v, ones_v, acc_sh,
            ssem):
    c = lax.axis_index("c")
    s = lax.axis_index("s")
    w = s * 2 + c
    pltpu.sync_copy(zeros_hbm, acc_sh.at[pl.ds(s * _RPT, _RPT)])
    pltpu.sync_copy(dst_hbm.at[w], idx_v)
    pltpu.sync_copy(ones_hbm, ones_v)
    plsc.subcore_barrier()

    def body(j, carry):
        pltpu.async_copy(ones_v, acc_sh.at[idx_v.at[j]], ssem, add=True)

        @pl.when(j >= 4)
        def _():
            pltpu.make_async_copy(ones_v, acc_sh.at[idx_v.at[0]],
                                  ssem).wait()

        return carry

    lax.fori_loop(0, _KCH, body, 0)

    def drain(j, carry):
        pltpu.make_async_copy(ones_v, acc_sh.at[idx_v.at[0]], ssem).wait()
        return carry

    lax.fori_loop(0, 4, drain, 0)
    plsc.subcore_barrier()
    pltpu.sync_copy(acc_sh.at[pl.ds(s * _RPT, _RPT)],
                    out_hbm.at[c, pl.ds(s * _RPT, _RPT)])


_IB = 16


@functools.partial(
    pl.kernel,
    out_type=jax.ShapeDtypeStruct((2, _NPAD, 128), jnp.float32),
    mesh=_sc_mesh(),
    scratch_types=[
        pltpu.VMEM((2, _IB, 2, _CH), jnp.int32),
        pltpu.VMEM((2, _CH, 128), jnp.float32),
        pltpu.VMEM_SHARED((_NPAD, 128), jnp.float32),
        pltpu.SemaphoreType.DMA,
        pltpu.SemaphoreType.DMA,
    ],
    compiler_params=pltpu.CompilerParams(use_tc_tiling_on_sc=False),
)
def _scatter128(g_hbm, ei_hbm, zeros_hbm, out_hbm, idx_v, rows_v, acc_sh,
                gs0, gs1):
    c = lax.axis_index("c")
    s = lax.axis_index("s")
    pltpu.sync_copy(zeros_hbm, acc_sh.at[pl.ds(s * _RPT, _RPT)])
    plsc.subcore_barrier()

    base = s * _SPS + c * _K0
    myk = jnp.where(c == 0, _K0, _SPS - _K0)

    def body(j, carry):
        @pl.when(jnp.logical_and(j < myk, j % _IB == 0))
        def _():
            pltpu.sync_copy(ei_hbm.at[pl.ds(base + j, _IB)],
                            idx_v.at[(j // _IB) % 2])

        @pl.when(j < myk)
        def _():
            wj = (j // _IB) % 2
            sj = j % _IB

            @pl.when(j % 2 == 0)
            def _():
                pltpu.async_copy(g_hbm.at[idx_v.at[wj, sj, 0]],
                                 rows_v.at[0], gs0)

            @pl.when(j % 2 == 1)
            def _():
                pltpu.async_copy(g_hbm.at[idx_v.at[wj, sj, 0]],
                                 rows_v.at[1], gs1)

        @pl.when(j >= 1)
        def _():
            jm = j - 1
            wm = (jm // _IB) % 2
            sm = jm % _IB

            @pl.when(jm % 2 == 0)
            def _():
                pltpu.make_async_copy(g_hbm.at[idx_v.at[0, 0, 0]],
                                      rows_v.at[0], gs0).wait()

            @pl.when(jm % 2 == 1)
            def _():
                pltpu.make_async_copy(g_hbm.at[idx_v.at[0, 0, 0]],
                                      rows_v.at[1], gs1).wait()

            pltpu.sync_copy(rows_v.at[jm % 2], acc_sh.at[idx_v.at[wm, sm, 1]],
                            add=True)

        return carry

    lax.fori_loop(0, myk + 1, body, 0)
    plsc.subcore_barrier()
    pltpu.sync_copy(acc_sh.at[pl.ds(s * _RPT, _RPT)],
                    out_hbm.at[c, pl.ds(s * _RPT, _RPT)])


@functools.partial(
    pl.kernel,
    out_type=jax.ShapeDtypeStruct((2, _NPAD, 64), jnp.float32),
    mesh=_sc_mesh(),
    scratch_types=[
        pltpu.VMEM((2, _IB, 2, _CH), jnp.int32),
        pltpu.VMEM((2, _CH, 64), jnp.float32),
        pltpu.VMEM_SHARED((_NPAD, 64), jnp.float32),
        pltpu.SemaphoreType.DMA,
        pltpu.SemaphoreType.DMA,
    ],
    compiler_params=pltpu.CompilerParams(use_tc_tiling_on_sc=False),
)
def _scatter64(g_hbm, ei_hbm, zeros_hbm, out_hbm, idx_v, rows_v, acc_sh,
               gs0, gs1):
    c = lax.axis_index("c")
    s = lax.axis_index("s")
    pltpu.sync_copy(zeros_hbm, acc_sh.at[pl.ds(s * _RPT, _RPT)])
    plsc.subcore_barrier()

    base = s * _SPS + c * _K0
    myk = jnp.where(c == 0, _K0, _SPS - _K0)

    def body(j, carry):
        @pl.when(jnp.logical_and(j < myk, j % _IB == 0))
        def _():
            pltpu.sync_copy(ei_hbm.at[pl.ds(base + j, _IB)],
                            idx_v.at[(j // _IB) % 2])

        @pl.when(j < myk)
        def _():
            wj = (j // _IB) % 2
            sj = j % _IB

            @pl.when(j % 2 == 0)
            def _():
                pltpu.async_copy(g_hbm.at[idx_v.at[wj, sj, 0]],
                                 rows_v.at[0], gs0)

            @pl.when(j % 2 == 1)
            def _():
                pltpu.async_copy(g_hbm.at[idx_v.at[wj, sj, 0]],
                                 rows_v.at[1], gs1)

        @pl.when(j >= 1)
        def _():
            jm = j - 1
            wm = (jm // _IB) % 2
            sm = jm % _IB

            @pl.when(jm % 2 == 0)
            def _():
                pltpu.make_async_copy(g_hbm.at[idx_v.at[0, 0, 0]],
                                      rows_v.at[0], gs0).wait()

            @pl.when(jm % 2 == 1)
            def _():
                pltpu.make_async_copy(g_hbm.at[idx_v.at[0, 0, 0]],
                                      rows_v.at[1], gs1).wait()

            pltpu.sync_copy(rows_v.at[jm % 2], acc_sh.at[idx_v.at[wm, sm, 1]],
                            add=True)

        return carry

    lax.fori_loop(0, myk + 1, body, 0)
    plsc.subcore_barrier()
    pltpu.sync_copy(acc_sh.at[pl.ds(s * _RPT, _RPT)],
                    out_hbm.at[c, pl.ds(s * _RPT, _RPT)])


def _tca_body(x_ref, w1_ref, degp_ref, g1_ref, dinv_ref):
    deg = degp_ref[0, :, 0:1] + degp_ref[1, :, 0:1] + 1.0
    dinv = lax.rsqrt(deg)
    h = jnp.dot(x_ref[...], w1_ref[...], preferred_element_type=jnp.float32)
    g1_ref[...] = dinv * h
    dinv_ref[...] = jnp.broadcast_to(dinv, dinv_ref.shape)


_BLK = 632


def _tc_a(x_pad, W1, degT):
    n_blk = _NPAD // _BLK
    return pl.pallas_call(
        _tca_body,
        grid=(n_blk,),
        in_specs=[
            pl.BlockSpec((_BLK, 128), lambda i: (i, 0)),
            pl.BlockSpec((128, 128), lambda i: (0, 0)),
            pl.BlockSpec((2, _BLK, 128), lambda i: (0, i, 0)),
        ],
        out_specs=[
            pl.BlockSpec((_BLK, 128), lambda i: (i, 0)),
            pl.BlockSpec((_BLK, 8), lambda i: (i, 0)),
        ],
        out_shape=[
            jax.ShapeDtypeStruct((_NPAD, 128), jnp.float32),
            jax.ShapeDtypeStruct((_NPAD, 8), jnp.float32),
        ],
    )(x_pad, W1, degT)


def _tcb_body(acc_ref, g1_ref, dinv_ref, b1_ref, lw1_ref, lb1_ref,
              lw2_ref, lb2_ref, w2_ref, g2_ref):
    dinv = dinv_ref[:, 0:1]
    s = acc_ref[0] + acc_ref[1] + g1_ref[...]
    out1 = jnp.maximum(dinv * s + b1_ref[...], 0.0)
    h2 = jnp.maximum(
        jnp.dot(out1, lw1_ref[...], preferred_element_type=jnp.float32)
        + lb1_ref[...], 0.0)
    h3 = jnp.maximum(
        jnp.dot(h2, lw2_ref[...], preferred_element_type=jnp.float32)
        + lb2_ref[...], 0.0)
    g2_ref[...] = dinv * jnp.dot(h3, w2_ref[...],
                                 preferred_element_type=jnp.float32)


def _tc_b(acc1, g1, dinv8, b1, Lw1, Lb1, Lw2, Lb2, W2):
    n_blk = _NPAD // _BLK
    return pl.pallas_call(
        _tcb_body,
        grid=(n_blk,),
        in_specs=[
            pl.BlockSpec((2, _BLK, 128), lambda i: (0, i, 0)),
            pl.BlockSpec((_BLK, 128), lambda i: (i, 0)),
            pl.BlockSpec((_BLK, 8), lambda i: (i, 0)),
            pl.BlockSpec((1, 128), lambda i: (0, 0)),
            pl.BlockSpec((128, 128), lambda i: (0, 0)),
            pl.BlockSpec((1, 128), lambda i: (0, 0)),
            pl.BlockSpec((128, 128), lambda i: (0, 0)),
            pl.BlockSpec((1, 128), lambda i: (0, 0)),
            pl.BlockSpec((128, 64), lambda i: (0, 0)),
        ],
        out_specs=pl.BlockSpec((_BLK, 64), lambda i: (i, 0)),
        out_shape=jax.ShapeDtypeStruct((_NPAD, 64), jnp.float32),
    )(acc1, g1, dinv8, b1, Lw1, Lb1, Lw2, Lb2, W2)


def _tcc_body(acc_ref, g2_ref, dinv_ref, b2_ref, out_ref):
    dinv = dinv_ref[:, 0:1]
    s = acc_ref[0] + acc_ref[1] + g2_ref[...]
    out_ref[...] = dinv * s + b2_ref[...]


_CBLK = 2000


def _tc_c(acc2, g2, dinv8, b2):
    n_blk = _N // _CBLK
    return pl.pallas_call(
        _tcc_body,
        grid=(n_blk,),
        in_specs=[
            pl.BlockSpec((2, _CBLK, 64), lambda i: (0, i, 0)),
            pl.BlockSpec((_CBLK, 64), lambda i: (i, 0)),
            pl.BlockSpec((_CBLK, 8), lambda i: (i, 0)),
            pl.BlockSpec((1, 64), lambda i: (0, 0)),
        ],
        out_specs=pl.BlockSpec((_CBLK, 64), lambda i: (i, 0)),
        out_shape=jax.ShapeDtypeStruct((_N, 64), jnp.float32),
    )(acc2, g2, dinv8, b2)


def kernel(x, edge_index, W1, b1, Lw1, Lb1, Lw2, Lb2, W2, b2):
    src = edge_index[0]
    dst = edge_index[1]
    pad = jnp.full((_EPAD - _E,), _N, jnp.int32)
    srcf = jnp.concatenate([src, pad])
    dstf = jnp.concatenate([dst, pad])
    dstp = dstf.reshape(_NW, _KCH, _CH)
    ei = jnp.stack([srcf.reshape(_TCH, _CH), dstf.reshape(_TCH, _CH)],
                   axis=1)
    x_pad = jnp.zeros((_NPAD, 128), jnp.float32).at[:_N].set(x)

    ones128 = jnp.ones((_CH, 128), jnp.float32)
    zeros128 = jnp.zeros((_RPT, 128), jnp.float32)

    degp = _deg_sc(dstp, ones128, zeros128)
    g1, dinv8 = _tc_a(x_pad, W1, degp)
    acc1 = _scatter128(g1, ei, zeros128)
    g2 = _tc_b(acc1, g1, dinv8, b1.reshape(1, 128), Lw1,
               Lb1.reshape(1, 128), Lw2, Lb2.reshape(1, 128), W2)
    zeros64 = jnp.zeros((_RPT, 64), jnp.float32)
    acc2 = _scatter64(g2, ei, zeros64)
    return _tc_c(acc2, g2, dinv8, b2.reshape(1, 64))
